# Initial kernel scaffold; baseline (speedup 1.0000x reference)
import jax
import jax.numpy as jnp
from jax.experimental import pallas as pl


def kernel(predicted_iv, node_embeddings, geometry, liquidity, edge_index_strike, edge_index_maturity):
    raise NotImplementedError("write your pallas kernel here")


# speedup vs baseline: 1.0866x; 1.0866x over previous
import functools

import jax
import jax.numpy as jnp
from jax.experimental import pallas as pl

N = 4096
D = 64
E = 16384
EPS = 1e-06
CAL_W = 1.0
BF_W = 1.0
PCP_W = 0.5
CVX_W = 0.5
LIQ_SCALE = 1.0

_RBLK = 512


def _tables_body(ivT, geomT, liqT, embT, out_ref):
    lm = geomT[0:1, :]
    tau = geomT[1:2, :]
    cp = geomT[2:3, :]
    iv = ivT[0:1, :]
    tv = iv * iv * jnp.maximum(tau, EPS)
    liq_w = jax.nn.sigmoid(LIQ_SCALE * liqT[0:1, :])
    callf = (cp > 0.5).astype(jnp.float32)
    nrm2 = jnp.sum(embT[...] * embT[...], axis=0, keepdims=True)
    enorm = jnp.sqrt(nrm2)
    out_ref[0:1, :] = tv
    out_ref[1:2, :] = liq_w
    out_ref[2:3, :] = callf
    out_ref[3:4, :] = enorm
    out_ref[4:5, :] = nrm2
    out_ref[5:6, :] = lm
    out_ref[6:7, :] = tau
    out_ref[7:8, :] = cp


def _argmin_body(rows_ref, colsT_ref, bp_ref):
    lm_r = rows_ref[:, 0:1]
    tau_r = rows_ref[:, 1:2]
    lm_c = colsT_ref[0:1, :]
    tau_c = colsT_ref[1:2, :]
    put_c = colsT_ref[2:3, :] <= 0.5
    d2 = (lm_r - lm_c) ** 2 + (tau_r - tau_c) ** 2
    d2m = jnp.where(put_c, d2, jnp.inf)
    bp = jnp.argmin(d2m, axis=1).astype(jnp.int32)
    bp_ref[0, 0, :] = bp


def kernel(predicted_iv, node_embeddings, geometry, liquidity, edge_index_strike, edge_index_maturity):
    geomT = geometry.T
    ivT = predicted_iv.T
    liqT = liquidity.reshape(1, N)
    embT = node_embeddings.T

    tables = pl.pallas_call(
        _tables_body,
        out_shape=jax.ShapeDtypeStruct((8, N), jnp.float32),
    )(ivT, geomT, liqT, embT)

    nblk = N // _RBLK
    best_put = pl.pallas_call(
        _argmin_body,
        grid=(nblk,),
        in_specs=[
            pl.BlockSpec((_RBLK, 3), lambda i: (i, 0)),
            pl.BlockSpec((3, N), lambda i: (0, 0)),
        ],
        out_specs=pl.BlockSpec((1, 1, _RBLK), lambda i: (i, 0, 0)),
        out_shape=jax.ShapeDtypeStruct((nblk, 1, _RBLK), jnp.int32),
    )(geometry, geomT).reshape(N)

    tv = tables[0]
    liq_w = tables[1]
    callf = tables[2]
    enorm = tables[3]
    nrm2 = tables[4]
    lm = tables[5]
    tau = tables[6]
    call_mask = callf > 0.5

    iv = predicted_iv[:, 0]
    Em = edge_index_strike.shape[1]
    src = edge_index_strike[0]
    dst = edge_index_strike[1]
    i_arr = jnp.concatenate([src, dst])
    nb_arr = jnp.concatenate([dst, src])
    rank = jnp.arange(2 * Em)
    valid = call_mask[i_arr] & call_mask[nb_arr]
    perm = jnp.lexsort((rank, lm[nb_arr], ~valid, i_arr))
    si = i_arr[perm]
    sn = nb_arr[perm]
    sv = valid[perm]
    n1 = sn[:-2]
    n2 = sn[1:-1]
    n3 = sn[2:]
    ii = si[:-2]
    tmask = (si[:-2] == si[1:-1]) & (si[1:-1] == si[2:]) & sv[:-2] & sv[1:-1] & sv[2:]
    cnt = jnp.sum(tmask.astype(jnp.float32))
    lm1 = lm[n1]
    lm2 = lm[n2]
    lm3 = lm[n3]
    denom = lm3 - lm1 + EPS
    alpha = (lm3 - lm2) / denom
    beta = (lm2 - lm1) / denom
    w1 = tv[n1]
    w2 = tv[n2]
    w3 = tv[n3]
    wgt = liq_w[ii] * liq_w[n2]
    bf_terms = wgt * jax.nn.relu(alpha * w1 + beta * w3 - w2)
    bf = jnp.sum(jnp.where(tmask, bf_terms, 0.0)) / jnp.maximum(cnt, 1.0)
    e1 = enorm[n1]
    e2 = enorm[n2]
    e3 = enorm[n3]
    cvx_terms = wgt * jax.nn.relu(e2 - (alpha * e1 + (1.0 - alpha) * e3))
    cvx = jnp.sum(jnp.where(tmask, cvx_terms, 0.0)) / jnp.maximum(cnt, 1.0)

    srcm = edge_index_maturity[0]
    dstm = edge_index_maturity[1]
    valid_cal = (tau[srcm] < tau[dstm] - EPS).astype(jnp.float32)
    cal_viol = jax.nn.relu(tv[srcm] - tv[dstm])
    cal_wgt = jnp.minimum(liq_w[srcm], liq_w[dstm])
    cal = jnp.sum(valid_cal * cal_wgt * cal_viol) / jnp.maximum(jnp.sum(valid_cal), 1.0)

    c_cnt = jnp.sum(callf)
    p_cnt = jnp.sum(1.0 - callf)
    emb_p = node_embeddings[best_put]
    dist = jnp.sum((node_embeddings - emb_p) ** 2, axis=1)
    pw = liq_w * liq_w[best_put]
    pcp_sum = jnp.sum(callf * pw * dist)
    pcp = jnp.where((c_cnt > 0) & (p_cnt > 0), pcp_sum / jnp.maximum(c_cnt, 1.0),
                    jnp.asarray(0.0, dtype=jnp.float32))
    return CAL_W * cal + BF_W * bf + PCP_W * pcp + CVX_W * cvx


# speedup vs baseline: 1.4098x; 1.2975x over previous
import functools

import jax
import jax.numpy as jnp
from jax.experimental import pallas as pl
from jax.experimental.pallas import tpu as pltpu

N = 4096
D = 64
E = 16384
EPS = 1e-06
CAL_W = 1.0
BF_W = 1.0
PCP_W = 0.5
CVX_W = 0.5
LIQ_SCALE = 1.0

_RBLK = 512
_M = 2 * E
_ROWS, _LANES = _M // 128, 128


def _partner_rows(x, jr):
    a = x.reshape(_ROWS // (2 * jr), 2, jr, _LANES)
    return jnp.concatenate([a[:, 1], a[:, 0]], axis=1).reshape(_ROWS, _LANES)


def _sort_body(k1_ref, k2_ref, r_ref, o1_ref, o2_ref, or_ref):
    k1 = k1_ref[...]
    k2 = k2_ref[...]
    rr = r_ref[...]
    row_i = jax.lax.broadcasted_iota(jnp.int32, (_ROWS, _LANES), 0)
    lane_i = jax.lax.broadcasted_iota(jnp.int32, (_ROWS, _LANES), 1)
    idx = row_i * _LANES + lane_i
    for m in range(1, 16):
        k = 1 << m
        asc = (idx & k) == 0
        for j in [1 << t for t in range(m - 1, -1, -1)]:
            if j >= _LANES:
                jr = j // _LANES
                p1 = _partner_rows(k1, jr)
                p2 = _partner_rows(k2, jr)
                pr = _partner_rows(rr, jr)
            else:
                lower = (lane_i & j) == 0
                jneg = _LANES - j
                p1 = jnp.where(lower, pltpu.roll(k1, jneg, 1), pltpu.roll(k1, j, 1))
                p2 = jnp.where(lower, pltpu.roll(k2, jneg, 1), pltpu.roll(k2, j, 1))
                pr = jnp.where(lower, pltpu.roll(rr, jneg, 1), pltpu.roll(rr, j, 1))
            gt = (k1 > p1) | ((k1 == p1) & ((k2 > p2) | ((k2 == p2) & (rr > pr))))
            keep_min = ((idx & j) == 0) == asc
            take_p = gt ^ (~keep_min)
            k1 = jnp.where(take_p, p1, k1)
            k2 = jnp.where(take_p, p2, k2)
            rr = jnp.where(take_p, pr, rr)
    o1_ref[...] = k1
    o2_ref[...] = k2
    or_ref[...] = rr


def _sort32k(k1, k2, r):
    outs = pl.pallas_call(
        _sort_body,
        out_shape=[jax.ShapeDtypeStruct((_ROWS, _LANES), jnp.int32)] * 3,
    )(k1.reshape(_ROWS, _LANES), k2.reshape(_ROWS, _LANES), r.reshape(_ROWS, _LANES))
    return [o.reshape(_M) for o in outs]


def _tables_body(ivT, geomT, liqT, embT, out_ref):
    lm = geomT[0:1, :]
    tau = geomT[1:2, :]
    cp = geomT[2:3, :]
    iv = ivT[0:1, :]
    tv = iv * iv * jnp.maximum(tau, EPS)
    liq_w = jax.nn.sigmoid(LIQ_SCALE * liqT[0:1, :])
    callf = (cp > 0.5).astype(jnp.float32)
    nrm2 = jnp.sum(embT[...] * embT[...], axis=0, keepdims=True)
    enorm = jnp.sqrt(nrm2)
    out_ref[0:1, :] = tv
    out_ref[1:2, :] = liq_w
    out_ref[2:3, :] = callf
    out_ref[3:4, :] = enorm
    out_ref[4:5, :] = nrm2
    out_ref[5:6, :] = lm
    out_ref[6:7, :] = tau
    out_ref[7:8, :] = cp


def _argmin_body(rows_ref, colsT_ref, bp_ref):
    lm_r = rows_ref[:, 0:1]
    tau_r = rows_ref[:, 1:2]
    lm_c = colsT_ref[0:1, :]
    tau_c = colsT_ref[1:2, :]
    put_c = colsT_ref[2:3, :] <= 0.5
    d2 = (lm_r - lm_c) ** 2 + (tau_r - tau_c) ** 2
    d2m = jnp.where(put_c, d2, jnp.inf)
    bp = jnp.argmin(d2m, axis=1).astype(jnp.int32)
    bp_ref[0, 0, :] = bp


def kernel(predicted_iv, node_embeddings, geometry, liquidity, edge_index_strike, edge_index_maturity):
    geomT = geometry.T
    ivT = predicted_iv.T
    liqT = liquidity.reshape(1, N)
    embT = node_embeddings.T

    tables = pl.pallas_call(
        _tables_body,
        out_shape=jax.ShapeDtypeStruct((8, N), jnp.float32),
    )(ivT, geomT, liqT, embT)

    nblk = N // _RBLK
    best_put = pl.pallas_call(
        _argmin_body,
        grid=(nblk,),
        in_specs=[
            pl.BlockSpec((_RBLK, 3), lambda i: (i, 0)),
            pl.BlockSpec((3, N), lambda i: (0, 0)),
        ],
        out_specs=pl.BlockSpec((1, 1, _RBLK), lambda i: (i, 0, 0)),
        out_shape=jax.ShapeDtypeStruct((nblk, 1, _RBLK), jnp.int32),
    )(geometry, geomT).reshape(N)

    tv = tables[0]
    liq_w = tables[1]
    callf = tables[2]
    enorm = tables[3]
    nrm2 = tables[4]
    lm = tables[5]
    tau = tables[6]
    call_mask = callf > 0.5

    iv = predicted_iv[:, 0]
    Em = edge_index_strike.shape[1]
    src = edge_index_strike[0]
    dst = edge_index_strike[1]
    i_arr = jnp.concatenate([src, dst])
    nb_arr = jnp.concatenate([dst, src])
    rank = jnp.arange(2 * Em)
    valid = call_mask[i_arr] & call_mask[nb_arr]
    key1 = 2 * i_arr.astype(jnp.int32) + (1 - valid.astype(jnp.int32))
    key2 = jax.lax.bitcast_convert_type(lm[nb_arr], jnp.int32)
    s1, s2, sr = _sort32k(key1, key2, rank.astype(jnp.int32))
    lms = jax.lax.bitcast_convert_type(s2, jnp.float32)
    sn = nb_arr[sr]
    si = s1 >> 1
    n1 = sn[:-2]
    n2 = sn[1:-1]
    n3 = sn[2:]
    ii = si[:-2]
    tmask = (s1[:-2] == s1[1:-1]) & (s1[1:-1] == s1[2:]) & (s1[:-2] % 2 == 0)
    cnt = jnp.sum(tmask.astype(jnp.float32))
    lm1 = lms[:-2]
    lm2 = lms[1:-1]
    lm3 = lms[2:]
    denom = lm3 - lm1 + EPS
    alpha = (lm3 - lm2) / denom
    beta = (lm2 - lm1) / denom
    w1 = tv[n1]
    w2 = tv[n2]
    w3 = tv[n3]
    wgt = liq_w[ii] * liq_w[n2]
    bf_terms = wgt * jax.nn.relu(alpha * w1 + beta * w3 - w2)
    bf = jnp.sum(jnp.where(tmask, bf_terms, 0.0)) / jnp.maximum(cnt, 1.0)
    e1 = enorm[n1]
    e2 = enorm[n2]
    e3 = enorm[n3]
    cvx_terms = wgt * jax.nn.relu(e2 - (alpha * e1 + (1.0 - alpha) * e3))
    cvx = jnp.sum(jnp.where(tmask, cvx_terms, 0.0)) / jnp.maximum(cnt, 1.0)

    srcm = edge_index_maturity[0]
    dstm = edge_index_maturity[1]
    valid_cal = (tau[srcm] < tau[dstm] - EPS).astype(jnp.float32)
    cal_viol = jax.nn.relu(tv[srcm] - tv[dstm])
    cal_wgt = jnp.minimum(liq_w[srcm], liq_w[dstm])
    cal = jnp.sum(valid_cal * cal_wgt * cal_viol) / jnp.maximum(jnp.sum(valid_cal), 1.0)

    c_cnt = jnp.sum(callf)
    p_cnt = jnp.sum(1.0 - callf)
    emb_p = node_embeddings[best_put]
    dist = jnp.sum((node_embeddings - emb_p) ** 2, axis=1)
    pw = liq_w * liq_w[best_put]
    pcp_sum = jnp.sum(callf * pw * dist)
    pcp = jnp.where((c_cnt > 0) & (p_cnt > 0), pcp_sum / jnp.maximum(c_cnt, 1.0),
                    jnp.asarray(0.0, dtype=jnp.float32))
    return CAL_W * cal + BF_W * bf + PCP_W * pcp + CVX_W * cvx


# speedup vs baseline: 52.7825x; 37.4390x over previous
import functools

import jax
import jax.numpy as jnp
from jax.experimental import pallas as pl
from jax.experimental.pallas import tpu as pltpu

N = 4096
D = 64
E = 16384
EPS = 1e-06
CAL_W = 1.0
BF_W = 1.0
PCP_W = 0.5
CVX_W = 0.5
LIQ_SCALE = 1.0

_RBLK = 512
_M = 2 * E
_ROWS, _LANES = _M // 128, 128


def _partner_rows(x, jr):
    a = x.reshape(_ROWS // (2 * jr), 2, jr, _LANES)
    return jnp.concatenate([a[:, 1], a[:, 0]], axis=1).reshape(_ROWS, _LANES)


def _sort_body(k1_ref, k2_ref, r_ref, o1_ref, o2_ref, or_ref):
    k1 = k1_ref[...]
    k2 = k2_ref[...]
    rr = r_ref[...]
    row_i = jax.lax.broadcasted_iota(jnp.int32, (_ROWS, _LANES), 0)
    lane_i = jax.lax.broadcasted_iota(jnp.int32, (_ROWS, _LANES), 1)
    idx = row_i * _LANES + lane_i
    for m in range(1, 16):
        k = 1 << m
        asc = (idx & k) == 0
        for j in [1 << t for t in range(m - 1, -1, -1)]:
            if j >= _LANES:
                jr = j // _LANES
                p1 = _partner_rows(k1, jr)
                p2 = _partner_rows(k2, jr)
                pr = _partner_rows(rr, jr)
            else:
                lower = (lane_i & j) == 0
                jneg = _LANES - j
                p1 = jnp.where(lower, pltpu.roll(k1, jneg, 1), pltpu.roll(k1, j, 1))
                p2 = jnp.where(lower, pltpu.roll(k2, jneg, 1), pltpu.roll(k2, j, 1))
                pr = jnp.where(lower, pltpu.roll(rr, jneg, 1), pltpu.roll(rr, j, 1))
            gt = (k1 > p1) | ((k1 == p1) & ((k2 > p2) | ((k2 == p2) & (rr > pr))))
            keep_min = ((idx & j) == 0) == asc
            take_p = gt ^ (~keep_min)
            k1 = jnp.where(take_p, p1, k1)
            k2 = jnp.where(take_p, p2, k2)
            rr = jnp.where(take_p, pr, rr)
    o1_ref[...] = k1
    o2_ref[...] = k2
    or_ref[...] = rr


def _sort32k(k1, k2, r):
    outs = pl.pallas_call(
        _sort_body,
        out_shape=[jax.ShapeDtypeStruct((_ROWS, _LANES), jnp.int32)] * 3,
    )(k1.reshape(_ROWS, _LANES), k2.reshape(_ROWS, _LANES), r.reshape(_ROWS, _LANES))
    return [o.reshape(_M) for o in outs]


def _tables_body(ivT, geomT, liqT, embT, out_ref):
    lm = geomT[0:1, :]
    tau = geomT[1:2, :]
    cp = geomT[2:3, :]
    iv = ivT[0:1, :]
    tv = iv * iv * jnp.maximum(tau, EPS)
    liq_w = jax.nn.sigmoid(LIQ_SCALE * liqT[0:1, :])
    callf = (cp > 0.5).astype(jnp.float32)
    nrm2 = jnp.sum(embT[...] * embT[...], axis=0, keepdims=True)
    enorm = jnp.sqrt(nrm2)
    out_ref[0:1, :] = tv
    out_ref[1:2, :] = liq_w
    out_ref[2:3, :] = callf
    out_ref[3:4, :] = enorm
    out_ref[4:5, :] = nrm2
    out_ref[5:6, :] = lm
    out_ref[6:7, :] = tau
    out_ref[7:8, :] = cp


def _argmin_body(rows_ref, colsT_ref, bp_ref):
    lm_r = rows_ref[:, 0:1]
    tau_r = rows_ref[:, 1:2]
    lm_c = colsT_ref[0:1, :]
    tau_c = colsT_ref[1:2, :]
    put_c = colsT_ref[2:3, :] <= 0.5
    d2 = (lm_r - lm_c) ** 2 + (tau_r - tau_c) ** 2
    d2m = jnp.where(put_c, d2, jnp.inf)
    bp = jnp.argmin(d2m, axis=1).astype(jnp.int32)
    bp_ref[0, 0, :] = bp


try:
    from jax.experimental.pallas import tpu_sc as plsc
    _HAVE_SC = True
except ImportError:
    _HAVE_SC = False

_NW = 32
_CHUNK = _M // _NW
_CAL_CHUNK = E // _NW
_NODE_CHUNK = N // _NW
_PAD = 128
_SENT = 0x7FFFFFFF


def _flat_tile_id():
    import jax.lax as lax
    return lax.axis_index("s") * 2 + lax.axis_index("c")


def _b1_body(i_hbm, nb_hbm, tab_hbm, eim_hbm, k1_hbm, k2_hbm, part_hbm,
             i_buf, nb_buf, cf_buf, lm_buf, tau_buf, tv_buf, lw_buf,
             k1_buf, k2_buf, sm_buf, dm_buf, out_buf):
    w = _flat_tile_id()
    base = w * _CHUNK
    pltpu.sync_copy(i_hbm.at[pl.ds(base, _CHUNK)], i_buf)
    pltpu.sync_copy(nb_hbm.at[pl.ds(base, _CHUNK)], nb_buf)
    pltpu.sync_copy(tab_hbm.at[2], cf_buf)
    pltpu.sync_copy(tab_hbm.at[5], lm_buf)
    pltpu.sync_copy(tab_hbm.at[6], tau_buf)
    pltpu.sync_copy(tab_hbm.at[0], tv_buf)
    pltpu.sync_copy(tab_hbm.at[1], lw_buf)
    cbase = w * _CAL_CHUNK
    pltpu.sync_copy(eim_hbm.at[0, pl.ds(cbase, _CAL_CHUNK)], sm_buf)
    pltpu.sync_copy(eim_hbm.at[1, pl.ds(cbase, _CAL_CHUNK)], dm_buf)

    def key_step(t, _):
        s = t * 16
        iv = i_buf[pl.ds(s, 16)]
        nv = nb_buf[pl.ds(s, 16)]
        ci = plsc.load_gather(cf_buf, [iv]) > 0.5
        cn = plsc.load_gather(cf_buf, [nv]) > 0.5
        inv = jnp.where(ci & cn, 0, 1)
        k1_buf[pl.ds(s, 16)] = iv * 2 + inv
        k2_buf[pl.ds(s, 16)] = plsc.bitcast(plsc.load_gather(lm_buf, [nv]),
                                            jnp.int32)
        return _

    jax.lax.fori_loop(0, _CHUNK // 16, key_step, 0)
    pltpu.sync_copy(k1_buf, k1_hbm.at[pl.ds(base, _CHUNK)])
    pltpu.sync_copy(k2_buf, k2_hbm.at[pl.ds(base, _CHUNK)])

    def cal_step(t, acc):
        acc_n, acc_c = acc
        s = t * 16
        sv = sm_buf[pl.ds(s, 16)]
        dv = dm_buf[pl.ds(s, 16)]
        ts = plsc.load_gather(tau_buf, [sv])
        td = plsc.load_gather(tau_buf, [dv])
        vcal = ts < td - EPS
        viol = jnp.maximum(plsc.load_gather(tv_buf, [sv])
                           - plsc.load_gather(tv_buf, [dv]), 0.0)
        wgt = jnp.minimum(plsc.load_gather(lw_buf, [sv]),
                          plsc.load_gather(lw_buf, [dv]))
        acc_n = acc_n + jnp.where(vcal, wgt * viol, 0.0)
        acc_c = acc_c + jnp.where(vcal, 1.0, 0.0)
        return acc_n, acc_c

    zero = jnp.zeros((16,), jnp.float32)
    cal_n, cal_c = jax.lax.fori_loop(0, _CAL_CHUNK // 16, cal_step, (zero, zero))
    out_buf[pl.ds(0, 16)] = cal_n
    out_buf[pl.ds(16, 16)] = cal_c
    pltpu.sync_copy(out_buf, part_hbm.at[w])


def _b2_body(s1_hbm, s2_hbm, sr_hbm, nb_hbm, tab_hbm, embT_hbm, bp_hbm,
             part_hbm,
             s1_buf, s2_buf, sr_buf, nb_buf, tv_buf, en_buf, lw_buf, cf_buf,
             bp_buf, ed0_buf, ed1_buf, w_buf, out_buf):
    w = _flat_tile_id()
    base = w * _CHUNK
    hl = _CHUNK + 16
    pltpu.sync_copy(s1_hbm.at[pl.ds(base, hl)], s1_buf)
    pltpu.sync_copy(s2_hbm.at[pl.ds(base, hl)], s2_buf)
    pltpu.sync_copy(sr_hbm.at[pl.ds(base, hl)], sr_buf)
    pltpu.sync_copy(nb_hbm, nb_buf)
    pltpu.sync_copy(tab_hbm.at[0], tv_buf)
    pltpu.sync_copy(tab_hbm.at[3], en_buf)
    pltpu.sync_copy(tab_hbm.at[1], lw_buf)
    pltpu.sync_copy(tab_hbm.at[2], cf_buf)
    pltpu.sync_copy(bp_hbm, bp_buf)
    pltpu.sync_copy(embT_hbm.at[2 * w], ed0_buf)
    pltpu.sync_copy(embT_hbm.at[2 * w + 1], ed1_buf)

    iota16 = jax.lax.iota(jnp.int32, 16)

    def tri_step(t, acc):
        acc_bf, acc_cv, acc_ct = acc
        s = t * 16
        a = s1_buf[pl.ds(s, 16)]
        b = s1_buf[pl.ds(s + 1, 16)]
        c = s1_buf[pl.ds(s + 2, 16)]
        lm1 = plsc.bitcast(s2_buf[pl.ds(s, 16)], jnp.float32)
        lm2 = plsc.bitcast(s2_buf[pl.ds(s + 1, 16)], jnp.float32)
        lm3 = plsc.bitcast(s2_buf[pl.ds(s + 2, 16)], jnp.float32)
        sn1 = plsc.load_gather(nb_buf, [sr_buf[pl.ds(s, 16)]])
        sn2 = plsc.load_gather(nb_buf, [sr_buf[pl.ds(s + 1, 16)]])
        sn3 = plsc.load_gather(nb_buf, [sr_buf[pl.ds(s + 2, 16)]])
        w1 = plsc.load_gather(tv_buf, [sn1])
        w2 = plsc.load_gather(tv_buf, [sn2])
        w3 = plsc.load_gather(tv_buf, [sn3])
        e1 = plsc.load_gather(en_buf, [sn1])
        e2 = plsc.load_gather(en_buf, [sn2])
        e3 = plsc.load_gather(en_buf, [sn3])
        lwn2 = plsc.load_gather(lw_buf, [sn2])
        lwi = plsc.load_gather(
            lw_buf, [jnp.minimum(jax.lax.shift_right_logical(a, 1), N - 1)])
        gj = base + s + iota16
        tmask = ((a == b) & (b == c) & ((a & 1) == 0) & (gj < (_M - 2)))
        denom = lm3 - lm1 + EPS
        alpha = (lm3 - lm2) / denom
        beta = (lm2 - lm1) / denom
        wgt = lwi * lwn2
        bft = wgt * jnp.maximum(alpha * w1 + beta * w3 - w2, 0.0)
        cvt = wgt * jnp.maximum(e2 - (alpha * e1 + (1.0 - alpha) * e3), 0.0)
        acc_bf = acc_bf + jnp.where(tmask, bft, 0.0)
        acc_cv = acc_cv + jnp.where(tmask, cvt, 0.0)
        acc_ct = acc_ct + jnp.where(tmask, 1.0, 0.0)
        return acc_bf, acc_cv, acc_ct

    zero = jnp.zeros((16,), jnp.float32)
    acc_bf, acc_cv, acc_ct = jax.lax.fori_loop(
        0, _CHUNK // 16, tri_step, (zero, zero, zero))

    def w_step(t, _):
        s = t * 16
        bpv = bp_buf[pl.ds(s, 16)]
        w_buf[pl.ds(s, 16)] = (cf_buf[pl.ds(s, 16)] * lw_buf[pl.ds(s, 16)]
                               * plsc.load_gather(lw_buf, [bpv]))
        return _

    jax.lax.fori_loop(0, N // 16, w_step, 0)

    def pcp_step(t, acc):
        s = t * 16
        bpv = bp_buf[pl.ds(s, 16)]
        wv = w_buf[pl.ds(s, 16)]
        d0 = ed0_buf[pl.ds(s, 16)] - plsc.load_gather(ed0_buf, [bpv])
        d1 = ed1_buf[pl.ds(s, 16)] - plsc.load_gather(ed1_buf, [bpv])
        return acc + wv * (d0 * d0 + d1 * d1)

    acc_pcp = jax.lax.fori_loop(0, N // 16, pcp_step, zero)

    out_buf[pl.ds(0, 16)] = acc_bf
    out_buf[pl.ds(16, 16)] = acc_cv
    out_buf[pl.ds(32, 16)] = acc_ct
    out_buf[pl.ds(48, 16)] = acc_pcp
    pltpu.sync_copy(out_buf, part_hbm.at[w])


def _run_b1(i_arr, nb_arr, tables, eim):
    mesh = plsc.VectorSubcoreMesh(core_axis_name="c", subcore_axis_name="s",
                                  num_cores=2, num_subcores=16)
    f = pl.kernel(
        _b1_body,
        out_type=[
            jax.ShapeDtypeStruct((_M,), jnp.int32),
            jax.ShapeDtypeStruct((_M,), jnp.int32),
            jax.ShapeDtypeStruct((_NW, 32), jnp.float32),
        ],
        mesh=mesh,
        compiler_params=pltpu.CompilerParams(needs_layout_passes=False),
        scratch_types=[
            pltpu.VMEM((_CHUNK,), jnp.int32),
            pltpu.VMEM((_CHUNK,), jnp.int32),
            pltpu.VMEM((N,), jnp.float32),
            pltpu.VMEM((N,), jnp.float32),
            pltpu.VMEM((N,), jnp.float32),
            pltpu.VMEM((N,), jnp.float32),
            pltpu.VMEM((N,), jnp.float32),
            pltpu.VMEM((_CHUNK,), jnp.int32),
            pltpu.VMEM((_CHUNK,), jnp.int32),
            pltpu.VMEM((_CAL_CHUNK,), jnp.int32),
            pltpu.VMEM((_CAL_CHUNK,), jnp.int32),
            pltpu.VMEM((32,), jnp.float32),
        ],
    )
    return f(i_arr, nb_arr, tables, eim)


def _run_b2(s1p, s2p, srp, nb_arr, tables, embT, bp):
    mesh = plsc.VectorSubcoreMesh(core_axis_name="c", subcore_axis_name="s",
                                  num_cores=2, num_subcores=16)
    f = pl.kernel(
        _b2_body,
        out_type=[jax.ShapeDtypeStruct((_NW, 64), jnp.float32)],
        mesh=mesh,
        compiler_params=pltpu.CompilerParams(needs_layout_passes=False),
        scratch_types=[
            pltpu.VMEM((_CHUNK + 16,), jnp.int32),
            pltpu.VMEM((_CHUNK + 16,), jnp.int32),
            pltpu.VMEM((_CHUNK + 16,), jnp.int32),
            pltpu.VMEM((_M,), jnp.int32),
            pltpu.VMEM((N,), jnp.float32),
            pltpu.VMEM((N,), jnp.float32),
            pltpu.VMEM((N,), jnp.float32),
            pltpu.VMEM((N,), jnp.float32),
            pltpu.VMEM((N,), jnp.int32),
            pltpu.VMEM((N,), jnp.float32),
            pltpu.VMEM((N,), jnp.float32),
            pltpu.VMEM((N,), jnp.float32),
            pltpu.VMEM((64,), jnp.float32),
        ],
    )
    return f(s1p, s2p, srp, nb_arr, tables, embT, bp)


def kernel(predicted_iv, node_embeddings, geometry, liquidity, edge_index_strike, edge_index_maturity):
    geomT = geometry.T
    ivT = predicted_iv.T
    liqT = liquidity.reshape(1, N)
    embT = node_embeddings.T

    tables = pl.pallas_call(
        _tables_body,
        out_shape=jax.ShapeDtypeStruct((8, N), jnp.float32),
    )(ivT, geomT, liqT, embT)

    nblk = N // _RBLK
    best_put = pl.pallas_call(
        _argmin_body,
        grid=(nblk,),
        in_specs=[
            pl.BlockSpec((_RBLK, 3), lambda i: (i, 0)),
            pl.BlockSpec((3, N), lambda i: (0, 0)),
        ],
        out_specs=pl.BlockSpec((1, 1, _RBLK), lambda i: (i, 0, 0)),
        out_shape=jax.ShapeDtypeStruct((nblk, 1, _RBLK), jnp.int32),
    )(geometry, geomT).reshape(N)

    tv = tables[0]
    liq_w = tables[1]
    callf = tables[2]
    enorm = tables[3]
    nrm2 = tables[4]
    lm = tables[5]
    tau = tables[6]
    call_mask = callf > 0.5

    src = edge_index_strike[0].astype(jnp.int32)
    dst = edge_index_strike[1].astype(jnp.int32)
    i_arr = jnp.concatenate([src, dst])
    nb_arr = jnp.concatenate([dst, src])
    eim = edge_index_maturity.astype(jnp.int32)

    key1, key2, cal_part = _run_b1(i_arr, nb_arr, tables, eim)
    s1, s2, sr = _sort32k(key1, key2, jnp.arange(_M, dtype=jnp.int32))
    s1p = jnp.concatenate([s1, jnp.full((_PAD,), _SENT, jnp.int32)])
    s2p = jnp.concatenate([s2, jnp.zeros((_PAD,), jnp.int32)])
    srp = jnp.concatenate([sr, jnp.zeros((_PAD,), jnp.int32)])
    embT_f = embT
    (b2_part,) = _run_b2(s1p, s2p, srp, nb_arr, tables, embT_f, best_put)

    cal_num = jnp.sum(cal_part[:, 0:16])
    cal_cnt = jnp.sum(cal_part[:, 16:32])
    cal = cal_num / jnp.maximum(cal_cnt, 1.0)
    bf_num = jnp.sum(b2_part[:, 0:16])
    cvx_num = jnp.sum(b2_part[:, 16:32])
    cnt = jnp.sum(b2_part[:, 32:48])
    pcp_sum = jnp.sum(b2_part[:, 48:64])
    bf = bf_num / jnp.maximum(cnt, 1.0)
    cvx = cvx_num / jnp.maximum(cnt, 1.0)

    c_cnt = jnp.sum(callf)
    p_cnt = N - c_cnt
    pcp = jnp.where((c_cnt > 0) & (p_cnt > 0), pcp_sum / jnp.maximum(c_cnt, 1.0),
                    jnp.asarray(0.0, dtype=jnp.float32))
    return CAL_W * cal + BF_W * bf + PCP_W * pcp + CVX_W * cvx


# speedup vs baseline: 60.0303x; 1.1373x over previous
import functools

import jax
import jax.numpy as jnp
from jax.experimental import pallas as pl
from jax.experimental.pallas import tpu as pltpu

N = 4096
D = 64
E = 16384
EPS = 1e-06
CAL_W = 1.0
BF_W = 1.0
PCP_W = 0.5
CVX_W = 0.5
LIQ_SCALE = 1.0

_RBLK = 512
_M = 2 * E
_ROWS, _LANES = _M // 128, 128


def _partner_rows(x, jr):
    a = x.reshape(_ROWS // (2 * jr), 2, jr, _LANES)
    return jnp.concatenate([a[:, 1], a[:, 0]], axis=1).reshape(_ROWS, _LANES)


def _sort_body(ka_ref, kb_ref, oa_ref, ob_ref):
    ka = ka_ref[...]
    kb = kb_ref[...]
    row_i = jax.lax.broadcasted_iota(jnp.int32, (_ROWS, _LANES), 0)
    lane_i = jax.lax.broadcasted_iota(jnp.int32, (_ROWS, _LANES), 1)
    idx = row_i * _LANES + lane_i
    for m in range(1, 16):
        k = 1 << m
        asc = (idx & k) == 0
        for j in [1 << t for t in range(m - 1, -1, -1)]:
            if j >= _LANES:
                jr = j // _LANES
                pa = _partner_rows(ka, jr)
                pb = _partner_rows(kb, jr)
            else:
                lower = (lane_i & j) == 0
                jneg = _LANES - j
                pa = jnp.where(lower, pltpu.roll(ka, jneg, 1), pltpu.roll(ka, j, 1))
                pb = jnp.where(lower, pltpu.roll(kb, jneg, 1), pltpu.roll(kb, j, 1))
            ahi = ka >> 15
            phi = pa >> 15
            gt = (ahi > phi) | ((ahi == phi) & ((kb > pb) | ((kb == pb) & (ka > pa))))
            keep_min = ((idx & j) == 0) == asc
            take_p = gt ^ (~keep_min)
            ka = jnp.where(take_p, pa, ka)
            kb = jnp.where(take_p, pb, kb)
    oa_ref[0:_ROWS, :] = ka
    ob_ref[0:_ROWS, :] = kb
    oa_ref[_ROWS:_ROWS + 1, :] = jnp.full((1, _LANES), _SENT, jnp.int32)
    ob_ref[_ROWS:_ROWS + 1, :] = jnp.zeros((1, _LANES), jnp.int32)


def _sort32k(ka, kb):
    outs = pl.pallas_call(
        _sort_body,
        out_shape=[jax.ShapeDtypeStruct((_ROWS + 1, _LANES), jnp.int32)] * 2,
    )(ka.reshape(_ROWS, _LANES), kb.reshape(_ROWS, _LANES))
    return [o.reshape(_M + _LANES) for o in outs]


def _tables_body(ivT, geomT, liqT, embT, out_ref):
    lm = geomT[0:1, :]
    tau = geomT[1:2, :]
    cp = geomT[2:3, :]
    iv = ivT[0:1, :]
    tv = iv * iv * jnp.maximum(tau, EPS)
    liq_w = jax.nn.sigmoid(LIQ_SCALE * liqT[0:1, :])
    callf = (cp > 0.5).astype(jnp.float32)
    nrm2 = jnp.sum(embT[...] * embT[...], axis=0, keepdims=True)
    enorm = jnp.sqrt(nrm2)
    out_ref[0:1, :] = tv
    out_ref[1:2, :] = liq_w
    out_ref[2:3, :] = callf
    out_ref[3:4, :] = enorm
    out_ref[4:5, :] = nrm2
    out_ref[5:6, :] = lm
    out_ref[6:7, :] = tau
    out_ref[7:8, :] = jnp.full((1, N), jnp.sum(callf), jnp.float32)


def _argmin_body(rows_ref, colsT_ref, bp_ref):
    lm_r = rows_ref[:, 0:1]
    tau_r = rows_ref[:, 1:2]
    lm_c = colsT_ref[0:1, :]
    tau_c = colsT_ref[1:2, :]
    put_c = colsT_ref[2:3, :] <= 0.5
    d2 = (lm_r - lm_c) ** 2 + (tau_r - tau_c) ** 2
    d2m = jnp.where(put_c, d2, jnp.inf)
    bp = jnp.argmin(d2m, axis=1).astype(jnp.int32)
    bp_ref[0, 0, :] = bp


try:
    from jax.experimental.pallas import tpu_sc as plsc
    _HAVE_SC = True
except ImportError:
    _HAVE_SC = False

_NW = 32
_CHUNK = _M // _NW
_CAL_CHUNK = E // _NW
_NODE_CHUNK = N // _NW
_PAD = 128
_SENT = 0x7FFFFFFF


def _flat_tile_id():
    import jax.lax as lax
    return lax.axis_index("s") * 2 + lax.axis_index("c")


def _b1_body(eis_hbm, tab_hbm, eim_hbm, k1_hbm, k2_hbm, nbo_hbm, part_hbm,
             i_buf, nb_buf, cf_buf, lm_buf, tau_buf, tv_buf, lw_buf,
             k1_buf, k2_buf, sm_buf, dm_buf, out_buf):
    w = _flat_tile_id()
    base = w * _CHUNK
    half = w // 16
    off = (w % 16) * _CHUNK
    pltpu.sync_copy(eis_hbm.at[half, pl.ds(off, _CHUNK)], i_buf)
    pltpu.sync_copy(eis_hbm.at[1 - half, pl.ds(off, _CHUNK)], nb_buf)
    pltpu.sync_copy(nb_buf, nbo_hbm.at[pl.ds(base, _CHUNK)])
    pltpu.sync_copy(tab_hbm.at[2], cf_buf)
    pltpu.sync_copy(tab_hbm.at[5], lm_buf)
    pltpu.sync_copy(tab_hbm.at[6], tau_buf)
    pltpu.sync_copy(tab_hbm.at[0], tv_buf)
    pltpu.sync_copy(tab_hbm.at[1], lw_buf)
    cbase = w * _CAL_CHUNK
    pltpu.sync_copy(eim_hbm.at[0, pl.ds(cbase, _CAL_CHUNK)], sm_buf)
    pltpu.sync_copy(eim_hbm.at[1, pl.ds(cbase, _CAL_CHUNK)], dm_buf)

    iota16 = jax.lax.iota(jnp.int32, 16)

    def key_step(t, _):
        s = t * 16
        iv = i_buf[pl.ds(s, 16)]
        nv = nb_buf[pl.ds(s, 16)]
        ci = plsc.load_gather(cf_buf, [iv]) > 0.5
        cn = plsc.load_gather(cf_buf, [nv]) > 0.5
        inv = jnp.where(ci & cn, 0, 1)
        rank = base + s + iota16
        k1_buf[pl.ds(s, 16)] = ((iv * 2 + inv) << 15) | rank
        k2_buf[pl.ds(s, 16)] = plsc.bitcast(plsc.load_gather(lm_buf, [nv]),
                                            jnp.int32)
        return _

    jax.lax.fori_loop(0, _CHUNK // 16, key_step, 0)
    pltpu.sync_copy(k1_buf, k1_hbm.at[pl.ds(base, _CHUNK)])
    pltpu.sync_copy(k2_buf, k2_hbm.at[pl.ds(base, _CHUNK)])

    def cal_step(t, acc):
        acc_n, acc_c = acc
        s = t * 16
        sv = sm_buf[pl.ds(s, 16)]
        dv = dm_buf[pl.ds(s, 16)]
        ts = plsc.load_gather(tau_buf, [sv])
        td = plsc.load_gather(tau_buf, [dv])
        vcal = ts < td - EPS
        viol = jnp.maximum(plsc.load_gather(tv_buf, [sv])
                           - plsc.load_gather(tv_buf, [dv]), 0.0)
        wgt = jnp.minimum(plsc.load_gather(lw_buf, [sv]),
                          plsc.load_gather(lw_buf, [dv]))
        acc_n = acc_n + jnp.where(vcal, wgt * viol, 0.0)
        acc_c = acc_c + jnp.where(vcal, 1.0, 0.0)
        return acc_n, acc_c

    zero = jnp.zeros((16,), jnp.float32)
    cal_n, cal_c = jax.lax.fori_loop(0, _CAL_CHUNK // 16, cal_step, (zero, zero))
    out_buf[pl.ds(0, 16)] = cal_n
    out_buf[pl.ds(16, 16)] = cal_c
    pltpu.sync_copy(out_buf, part_hbm.at[w])


def _b2_body(s1_hbm, s2_hbm, nb_hbm, tab_hbm, embT_hbm, bp_hbm,
             part_hbm,
             s1_buf, s2_buf, nb_buf, tv_buf, en_buf, lw_buf, cf_buf,
             bp_buf, ed0_buf, ed1_buf, w_buf, out_buf):
    w = _flat_tile_id()
    base = w * _CHUNK
    hl = _CHUNK + 16
    pltpu.sync_copy(s1_hbm.at[pl.ds(base, hl)], s1_buf)
    pltpu.sync_copy(s2_hbm.at[pl.ds(base, hl)], s2_buf)
    pltpu.sync_copy(nb_hbm, nb_buf)
    pltpu.sync_copy(tab_hbm.at[0], tv_buf)
    pltpu.sync_copy(tab_hbm.at[3], en_buf)
    pltpu.sync_copy(tab_hbm.at[1], lw_buf)
    pltpu.sync_copy(tab_hbm.at[2], cf_buf)
    pltpu.sync_copy(bp_hbm, bp_buf)
    pltpu.sync_copy(embT_hbm.at[2 * w], ed0_buf)
    pltpu.sync_copy(embT_hbm.at[2 * w + 1], ed1_buf)

    iota16 = jax.lax.iota(jnp.int32, 16)

    def tri_step(t, acc):
        acc_bf, acc_cv, acc_ct = acc
        s = t * 16
        a = s1_buf[pl.ds(s, 16)]
        b = s1_buf[pl.ds(s + 1, 16)]
        c = s1_buf[pl.ds(s + 2, 16)]
        lm1 = plsc.bitcast(s2_buf[pl.ds(s, 16)], jnp.float32)
        lm2 = plsc.bitcast(s2_buf[pl.ds(s + 1, 16)], jnp.float32)
        lm3 = plsc.bitcast(s2_buf[pl.ds(s + 2, 16)], jnp.float32)
        sn1 = plsc.load_gather(nb_buf, [a & 32767])
        sn2 = plsc.load_gather(nb_buf, [b & 32767])
        sn3 = plsc.load_gather(nb_buf, [c & 32767])
        w1 = plsc.load_gather(tv_buf, [sn1])
        w2 = plsc.load_gather(tv_buf, [sn2])
        w3 = plsc.load_gather(tv_buf, [sn3])
        e1 = plsc.load_gather(en_buf, [sn1])
        e2 = plsc.load_gather(en_buf, [sn2])
        e3 = plsc.load_gather(en_buf, [sn3])
        lwn2 = plsc.load_gather(lw_buf, [sn2])
        lwi = plsc.load_gather(
            lw_buf, [jnp.minimum(jax.lax.shift_right_logical(a, 16), N - 1)])
        ahi = a >> 15
        bhi = b >> 15
        chi = c >> 15
        gj = base + s + iota16
        tmask = ((ahi == bhi) & (bhi == chi) & ((ahi & 1) == 0)
                 & (gj < (_M - 2)))
        denom = lm3 - lm1 + EPS
        alpha = (lm3 - lm2) / denom
        beta = (lm2 - lm1) / denom
        wgt = lwi * lwn2
        bft = wgt * jnp.maximum(alpha * w1 + beta * w3 - w2, 0.0)
        cvt = wgt * jnp.maximum(e2 - (alpha * e1 + (1.0 - alpha) * e3), 0.0)
        acc_bf = acc_bf + jnp.where(tmask, bft, 0.0)
        acc_cv = acc_cv + jnp.where(tmask, cvt, 0.0)
        acc_ct = acc_ct + jnp.where(tmask, 1.0, 0.0)
        return acc_bf, acc_cv, acc_ct

    zero = jnp.zeros((16,), jnp.float32)
    acc_bf, acc_cv, acc_ct = jax.lax.fori_loop(
        0, _CHUNK // 16, tri_step, (zero, zero, zero))

    def w_step(t, _):
        s = t * 16
        bpv = bp_buf[pl.ds(s, 16)]
        w_buf[pl.ds(s, 16)] = (cf_buf[pl.ds(s, 16)] * lw_buf[pl.ds(s, 16)]
                               * plsc.load_gather(lw_buf, [bpv]))
        return _

    jax.lax.fori_loop(0, N // 16, w_step, 0)

    def pcp_step(t, acc):
        s = t * 16
        bpv = bp_buf[pl.ds(s, 16)]
        wv = w_buf[pl.ds(s, 16)]
        d0 = ed0_buf[pl.ds(s, 16)] - plsc.load_gather(ed0_buf, [bpv])
        d1 = ed1_buf[pl.ds(s, 16)] - plsc.load_gather(ed1_buf, [bpv])
        return acc + wv * (d0 * d0 + d1 * d1)

    acc_pcp = jax.lax.fori_loop(0, N // 16, pcp_step, zero)

    out_buf[pl.ds(0, 16)] = acc_bf
    out_buf[pl.ds(16, 16)] = acc_cv
    out_buf[pl.ds(32, 16)] = acc_ct
    out_buf[pl.ds(48, 16)] = acc_pcp
    pltpu.sync_copy(out_buf, part_hbm.at[w])


def _run_b1(eis, tables, eim):
    mesh = plsc.VectorSubcoreMesh(core_axis_name="c", subcore_axis_name="s",
                                  num_cores=2, num_subcores=16)
    f = pl.kernel(
        _b1_body,
        out_type=[
            jax.ShapeDtypeStruct((_M,), jnp.int32),
            jax.ShapeDtypeStruct((_M,), jnp.int32),
            jax.ShapeDtypeStruct((_M,), jnp.int32),
            jax.ShapeDtypeStruct((_NW, 32), jnp.float32),
        ],
        mesh=mesh,
        compiler_params=pltpu.CompilerParams(needs_layout_passes=False),
        scratch_types=[
            pltpu.VMEM((_CHUNK,), jnp.int32),
            pltpu.VMEM((_CHUNK,), jnp.int32),
            pltpu.VMEM((N,), jnp.float32),
            pltpu.VMEM((N,), jnp.float32),
            pltpu.VMEM((N,), jnp.float32),
            pltpu.VMEM((N,), jnp.float32),
            pltpu.VMEM((N,), jnp.float32),
            pltpu.VMEM((_CHUNK,), jnp.int32),
            pltpu.VMEM((_CHUNK,), jnp.int32),
            pltpu.VMEM((_CAL_CHUNK,), jnp.int32),
            pltpu.VMEM((_CAL_CHUNK,), jnp.int32),
            pltpu.VMEM((32,), jnp.float32),
        ],
    )
    return f(eis, tables, eim)


def _run_b2(s1p, s2p, nb_arr, tables, embT, bp):
    mesh = plsc.VectorSubcoreMesh(core_axis_name="c", subcore_axis_name="s",
                                  num_cores=2, num_subcores=16)
    f = pl.kernel(
        _b2_body,
        out_type=[jax.ShapeDtypeStruct((_NW, 64), jnp.float32)],
        mesh=mesh,
        compiler_params=pltpu.CompilerParams(needs_layout_passes=False),
        scratch_types=[
            pltpu.VMEM((_CHUNK + 16,), jnp.int32),
            pltpu.VMEM((_CHUNK + 16,), jnp.int32),
            pltpu.VMEM((_M,), jnp.int32),
            pltpu.VMEM((N,), jnp.float32),
            pltpu.VMEM((N,), jnp.float32),
            pltpu.VMEM((N,), jnp.float32),
            pltpu.VMEM((N,), jnp.float32),
            pltpu.VMEM((N,), jnp.int32),
            pltpu.VMEM((N,), jnp.float32),
            pltpu.VMEM((N,), jnp.float32),
            pltpu.VMEM((N,), jnp.float32),
            pltpu.VMEM((64,), jnp.float32),
        ],
    )
    return f(s1p, s2p, nb_arr, tables, embT, bp)


def kernel(predicted_iv, node_embeddings, geometry, liquidity, edge_index_strike, edge_index_maturity):
    geomT = geometry.T
    ivT = predicted_iv.T
    liqT = liquidity.reshape(1, N)
    embT = node_embeddings.T

    tables = pl.pallas_call(
        _tables_body,
        out_shape=jax.ShapeDtypeStruct((8, N), jnp.float32),
    )(ivT, geomT, liqT, embT)

    nblk = N // _RBLK
    best_put = pl.pallas_call(
        _argmin_body,
        grid=(nblk,),
        in_specs=[
            pl.BlockSpec((_RBLK, 3), lambda i: (i, 0)),
            pl.BlockSpec((3, N), lambda i: (0, 0)),
        ],
        out_specs=pl.BlockSpec((1, 1, _RBLK), lambda i: (i, 0, 0)),
        out_shape=jax.ShapeDtypeStruct((nblk, 1, _RBLK), jnp.int32),
    )(geometry, geomT).reshape(N)

    tv = tables[0]
    liq_w = tables[1]
    callf = tables[2]
    enorm = tables[3]
    nrm2 = tables[4]
    lm = tables[5]
    tau = tables[6]
    call_mask = callf > 0.5

    eis = edge_index_strike.astype(jnp.int32)
    eim = edge_index_maturity.astype(jnp.int32)

    key1, key2, nb_arr, cal_part = _run_b1(eis, tables, eim)
    s1p, s2p = _sort32k(key1, key2)
    (b2_part,) = _run_b2(s1p, s2p, nb_arr, tables, embT, best_put)

    cal_num = jnp.sum(cal_part[:, 0:16])
    cal_cnt = jnp.sum(cal_part[:, 16:32])
    cal = cal_num / jnp.maximum(cal_cnt, 1.0)
    bf_num = jnp.sum(b2_part[:, 0:16])
    cvx_num = jnp.sum(b2_part[:, 16:32])
    cnt = jnp.sum(b2_part[:, 32:48])
    pcp_sum = jnp.sum(b2_part[:, 48:64])
    bf = bf_num / jnp.maximum(cnt, 1.0)
    cvx = cvx_num / jnp.maximum(cnt, 1.0)

    c_cnt = tables[7, 0]
    p_cnt = N - c_cnt
    pcp = jnp.where((c_cnt > 0) & (p_cnt > 0), pcp_sum / jnp.maximum(c_cnt, 1.0),
                    jnp.asarray(0.0, dtype=jnp.float32))
    return CAL_W * cal + BF_W * bf + PCP_W * pcp + CVX_W * cvx


# speedup vs baseline: 63.0308x; 1.0500x over previous
import functools

import jax
import jax.numpy as jnp
from jax.experimental import pallas as pl
from jax.experimental.pallas import tpu as pltpu

N = 4096
D = 64
E = 16384
EPS = 1e-06
CAL_W = 1.0
BF_W = 1.0
PCP_W = 0.5
CVX_W = 0.5
LIQ_SCALE = 1.0

_RBLK = 512
_M = 2 * E
_ROWS, _LANES = _M // 128, 128


def _partner_rows(x, jr):
    a = x.reshape(_ROWS // (2 * jr), 2, jr, _LANES)
    return jnp.concatenate([a[:, 1], a[:, 0]], axis=1).reshape(_ROWS, _LANES)


def _sort_body(ka_ref, kb_ref, oa_ref, ob_ref):
    ka = ka_ref[...]
    kb = kb_ref[...]
    row_i = jax.lax.broadcasted_iota(jnp.int32, (_ROWS, _LANES), 0)
    lane_i = jax.lax.broadcasted_iota(jnp.int32, (_ROWS, _LANES), 1)
    idx = row_i * _LANES + lane_i
    for m in range(1, 16):
        k = 1 << m
        asc = (idx & k) == 0
        for j in [1 << t for t in range(m - 1, -1, -1)]:
            if j >= _LANES:
                jr = j // _LANES
                pa = _partner_rows(ka, jr)
                pb = _partner_rows(kb, jr)
            else:
                lower = (lane_i & j) == 0
                jneg = _LANES - j
                pa = jnp.where(lower, pltpu.roll(ka, jneg, 1), pltpu.roll(ka, j, 1))
                pb = jnp.where(lower, pltpu.roll(kb, jneg, 1), pltpu.roll(kb, j, 1))
            ahi = ka >> 15
            phi = pa >> 15
            gt = (ahi > phi) | ((ahi == phi) & ((kb > pb) | ((kb == pb) & (ka > pa))))
            keep_min = ((idx & j) == 0) == asc
            take_p = gt ^ (~keep_min)
            ka = jnp.where(take_p, pa, ka)
            kb = jnp.where(take_p, pb, kb)
    oa_ref[0:_ROWS, :] = ka
    ob_ref[0:_ROWS, :] = kb
    oa_ref[_ROWS:_ROWS + 1, :] = jnp.full((1, _LANES), _SENT, jnp.int32)
    ob_ref[_ROWS:_ROWS + 1, :] = jnp.zeros((1, _LANES), jnp.int32)


def _sort32k(ka, kb):
    outs = pl.pallas_call(
        _sort_body,
        out_shape=[jax.ShapeDtypeStruct((_ROWS + 1, _LANES), jnp.int32)] * 2,
    )(ka.reshape(_ROWS, _LANES), kb.reshape(_ROWS, _LANES))
    return [o.reshape(_M + _LANES) for o in outs]


def _tables_body(ivT, geomT, liqT, embT, out_ref):
    lm = geomT[0:1, :]
    tau = geomT[1:2, :]
    cp = geomT[2:3, :]
    iv = ivT[0:1, :]
    tv = iv * iv * jnp.maximum(tau, EPS)
    liq_w = jax.nn.sigmoid(LIQ_SCALE * liqT[0:1, :])
    callf = (cp > 0.5).astype(jnp.float32)
    nrm2 = jnp.sum(embT[...] * embT[...], axis=0, keepdims=True)
    enorm = jnp.sqrt(nrm2)
    out_ref[0:1, :] = tv
    out_ref[1:2, :] = liq_w
    out_ref[2:3, :] = callf
    out_ref[3:4, :] = enorm
    out_ref[4:5, :] = nrm2
    out_ref[5:6, :] = lm
    out_ref[6:7, :] = tau
    out_ref[7:8, :] = jnp.full((1, N), jnp.sum(callf), jnp.float32)


def _argmin_body(rows_ref, colsT_ref, bp_ref):
    lm_r = rows_ref[:, 0:1]
    tau_r = rows_ref[:, 1:2]
    lm_c = colsT_ref[0:1, :]
    tau_c = colsT_ref[1:2, :]
    put_c = colsT_ref[2:3, :] <= 0.5
    d2 = (lm_r - lm_c) ** 2 + (tau_r - tau_c) ** 2
    d2m = jnp.where(put_c, d2, jnp.inf)
    bp = jnp.argmin(d2m, axis=1).astype(jnp.int32)
    bp_ref[0, 0, :] = bp


try:
    from jax.experimental.pallas import tpu_sc as plsc
    _HAVE_SC = True
except ImportError:
    _HAVE_SC = False

_NW = 32
_CHUNK = _M // _NW
_CAL_CHUNK = E // _NW
_NODE_CHUNK = N // _NW
_PAD = 128
_SENT = 0x7FFFFFFF


def _flat_tile_id():
    import jax.lax as lax
    return lax.axis_index("s") * 2 + lax.axis_index("c")


def _b1_body(eis_hbm, tab_hbm, eim_hbm, k1_hbm, k2_hbm, nbo_hbm, part_hbm,
             i_buf, nb_buf, cf_buf, lm_buf, tau_buf, tv_buf, lw_buf,
             k1_buf, k2_buf, sm_buf, dm_buf, out_buf, sem):
    w = _flat_tile_id()
    base = w * _CHUNK
    half = w // 16
    off = (w % 16) * _CHUNK
    cbase = w * _CAL_CHUNK
    copies = [
        pltpu.async_copy(eis_hbm.at[half, pl.ds(off, _CHUNK)], i_buf, sem),
        pltpu.async_copy(eis_hbm.at[1 - half, pl.ds(off, _CHUNK)], nb_buf, sem),
        pltpu.async_copy(tab_hbm.at[2], cf_buf, sem),
        pltpu.async_copy(tab_hbm.at[5], lm_buf, sem),
        pltpu.async_copy(tab_hbm.at[6], tau_buf, sem),
        pltpu.async_copy(tab_hbm.at[0], tv_buf, sem),
        pltpu.async_copy(tab_hbm.at[1], lw_buf, sem),
        pltpu.async_copy(eim_hbm.at[0, pl.ds(cbase, _CAL_CHUNK)], sm_buf, sem),
        pltpu.async_copy(eim_hbm.at[1, pl.ds(cbase, _CAL_CHUNK)], dm_buf, sem),
    ]
    for cp in copies:
        cp.wait()
    pltpu.sync_copy(nb_buf, nbo_hbm.at[pl.ds(base, _CHUNK)])

    iota16 = jax.lax.iota(jnp.int32, 16)

    def key_step(t, _):
        s = t * 16
        iv = i_buf[pl.ds(s, 16)]
        nv = nb_buf[pl.ds(s, 16)]
        ci = plsc.load_gather(cf_buf, [iv]) > 0.5
        cn = plsc.load_gather(cf_buf, [nv]) > 0.5
        inv = jnp.where(ci & cn, 0, 1)
        rank = base + s + iota16
        k1_buf[pl.ds(s, 16)] = ((iv * 2 + inv) << 15) | rank
        k2_buf[pl.ds(s, 16)] = plsc.bitcast(plsc.load_gather(lm_buf, [nv]),
                                            jnp.int32)
        return _

    jax.lax.fori_loop(0, _CHUNK // 16, key_step, 0)
    pltpu.sync_copy(k1_buf, k1_hbm.at[pl.ds(base, _CHUNK)])
    pltpu.sync_copy(k2_buf, k2_hbm.at[pl.ds(base, _CHUNK)])

    def cal_step(t, acc):
        acc_n, acc_c = acc
        s = t * 16
        sv = sm_buf[pl.ds(s, 16)]
        dv = dm_buf[pl.ds(s, 16)]
        ts = plsc.load_gather(tau_buf, [sv])
        td = plsc.load_gather(tau_buf, [dv])
        vcal = ts < td - EPS
        viol = jnp.maximum(plsc.load_gather(tv_buf, [sv])
                           - plsc.load_gather(tv_buf, [dv]), 0.0)
        wgt = jnp.minimum(plsc.load_gather(lw_buf, [sv]),
                          plsc.load_gather(lw_buf, [dv]))
        acc_n = acc_n + jnp.where(vcal, wgt * viol, 0.0)
        acc_c = acc_c + jnp.where(vcal, 1.0, 0.0)
        return acc_n, acc_c

    zero = jnp.zeros((16,), jnp.float32)
    cal_n, cal_c = jax.lax.fori_loop(0, _CAL_CHUNK // 16, cal_step, (zero, zero))
    out_buf[pl.ds(0, 16)] = cal_n
    out_buf[pl.ds(16, 16)] = cal_c
    pltpu.sync_copy(out_buf, part_hbm.at[w])


def _b2_body(s1_hbm, s2_hbm, nb_hbm, tab_hbm, embT_hbm, bp_hbm,
             part_hbm,
             s1_buf, s2_buf, nb_buf, tv_buf, en_buf, lw_buf, cf_buf,
             bp_buf, ed0_buf, ed1_buf, w_buf, out_buf, sem):
    w = _flat_tile_id()
    base = w * _CHUNK
    hl = _CHUNK + 16
    copies = [
        pltpu.async_copy(s1_hbm.at[pl.ds(base, hl)], s1_buf, sem),
        pltpu.async_copy(s2_hbm.at[pl.ds(base, hl)], s2_buf, sem),
        pltpu.async_copy(nb_hbm, nb_buf, sem),
        pltpu.async_copy(tab_hbm.at[0], tv_buf, sem),
        pltpu.async_copy(tab_hbm.at[3], en_buf, sem),
        pltpu.async_copy(tab_hbm.at[1], lw_buf, sem),
        pltpu.async_copy(tab_hbm.at[2], cf_buf, sem),
        pltpu.async_copy(bp_hbm, bp_buf, sem),
        pltpu.async_copy(embT_hbm.at[2 * w], ed0_buf, sem),
        pltpu.async_copy(embT_hbm.at[2 * w + 1], ed1_buf, sem),
    ]
    for cp in copies:
        cp.wait()

    iota16 = jax.lax.iota(jnp.int32, 16)

    def tri_step(t, acc):
        acc_bf, acc_cv, acc_ct = acc
        s = t * 16
        a = s1_buf[pl.ds(s, 16)]
        b = s1_buf[pl.ds(s + 1, 16)]
        c = s1_buf[pl.ds(s + 2, 16)]
        lm1 = plsc.bitcast(s2_buf[pl.ds(s, 16)], jnp.float32)
        lm2 = plsc.bitcast(s2_buf[pl.ds(s + 1, 16)], jnp.float32)
        lm3 = plsc.bitcast(s2_buf[pl.ds(s + 2, 16)], jnp.float32)
        sn1 = plsc.load_gather(nb_buf, [a & 32767])
        sn2 = plsc.load_gather(nb_buf, [b & 32767])
        sn3 = plsc.load_gather(nb_buf, [c & 32767])
        w1 = plsc.load_gather(tv_buf, [sn1])
        w2 = plsc.load_gather(tv_buf, [sn2])
        w3 = plsc.load_gather(tv_buf, [sn3])
        e1 = plsc.load_gather(en_buf, [sn1])
        e2 = plsc.load_gather(en_buf, [sn2])
        e3 = plsc.load_gather(en_buf, [sn3])
        lwn2 = plsc.load_gather(lw_buf, [sn2])
        lwi = plsc.load_gather(
            lw_buf, [jnp.minimum(jax.lax.shift_right_logical(a, 16), N - 1)])
        ahi = a >> 15
        bhi = b >> 15
        chi = c >> 15
        gj = base + s + iota16
        tmask = ((ahi == bhi) & (bhi == chi) & ((ahi & 1) == 0)
                 & (gj < (_M - 2)))
        denom = lm3 - lm1 + EPS
        alpha = (lm3 - lm2) / denom
        beta = (lm2 - lm1) / denom
        wgt = lwi * lwn2
        bft = wgt * jnp.maximum(alpha * w1 + beta * w3 - w2, 0.0)
        cvt = wgt * jnp.maximum(e2 - (alpha * e1 + (1.0 - alpha) * e3), 0.0)
        acc_bf = acc_bf + jnp.where(tmask, bft, 0.0)
        acc_cv = acc_cv + jnp.where(tmask, cvt, 0.0)
        acc_ct = acc_ct + jnp.where(tmask, 1.0, 0.0)
        return acc_bf, acc_cv, acc_ct

    zero = jnp.zeros((16,), jnp.float32)
    acc_bf, acc_cv, acc_ct = jax.lax.fori_loop(
        0, _CHUNK // 16, tri_step, (zero, zero, zero))

    def w_step(t, _):
        s = t * 16
        bpv = bp_buf[pl.ds(s, 16)]
        w_buf[pl.ds(s, 16)] = (cf_buf[pl.ds(s, 16)] * lw_buf[pl.ds(s, 16)]
                               * plsc.load_gather(lw_buf, [bpv]))
        return _

    jax.lax.fori_loop(0, N // 16, w_step, 0)

    def pcp_step(t, acc):
        s = t * 16
        bpv = bp_buf[pl.ds(s, 16)]
        wv = w_buf[pl.ds(s, 16)]
        d0 = ed0_buf[pl.ds(s, 16)] - plsc.load_gather(ed0_buf, [bpv])
        d1 = ed1_buf[pl.ds(s, 16)] - plsc.load_gather(ed1_buf, [bpv])
        return acc + wv * (d0 * d0 + d1 * d1)

    acc_pcp = jax.lax.fori_loop(0, N // 16, pcp_step, zero)

    out_buf[pl.ds(0, 16)] = acc_bf
    out_buf[pl.ds(16, 16)] = acc_cv
    out_buf[pl.ds(32, 16)] = acc_ct
    out_buf[pl.ds(48, 16)] = acc_pcp
    pltpu.sync_copy(out_buf, part_hbm.at[w])


def _run_b1(eis, tables, eim):
    mesh = plsc.VectorSubcoreMesh(core_axis_name="c", subcore_axis_name="s",
                                  num_cores=2, num_subcores=16)
    f = pl.kernel(
        _b1_body,
        out_type=[
            jax.ShapeDtypeStruct((_M,), jnp.int32),
            jax.ShapeDtypeStruct((_M,), jnp.int32),
            jax.ShapeDtypeStruct((_M,), jnp.int32),
            jax.ShapeDtypeStruct((_NW, 32), jnp.float32),
        ],
        mesh=mesh,
        compiler_params=pltpu.CompilerParams(needs_layout_passes=False),
        scratch_types=[
            pltpu.VMEM((_CHUNK,), jnp.int32),
            pltpu.VMEM((_CHUNK,), jnp.int32),
            pltpu.VMEM((N,), jnp.float32),
            pltpu.VMEM((N,), jnp.float32),
            pltpu.VMEM((N,), jnp.float32),
            pltpu.VMEM((N,), jnp.float32),
            pltpu.VMEM((N,), jnp.float32),
            pltpu.VMEM((_CHUNK,), jnp.int32),
            pltpu.VMEM((_CHUNK,), jnp.int32),
            pltpu.VMEM((_CAL_CHUNK,), jnp.int32),
            pltpu.VMEM((_CAL_CHUNK,), jnp.int32),
            pltpu.VMEM((32,), jnp.float32),
            pltpu.SemaphoreType.DMA,
        ],
    )
    return f(eis, tables, eim)


def _run_b2(s1p, s2p, nb_arr, tables, embT, bp):
    mesh = plsc.VectorSubcoreMesh(core_axis_name="c", subcore_axis_name="s",
                                  num_cores=2, num_subcores=16)
    f = pl.kernel(
        _b2_body,
        out_type=[jax.ShapeDtypeStruct((_NW, 64), jnp.float32)],
        mesh=mesh,
        compiler_params=pltpu.CompilerParams(needs_layout_passes=False),
        scratch_types=[
            pltpu.VMEM((_CHUNK + 16,), jnp.int32),
            pltpu.VMEM((_CHUNK + 16,), jnp.int32),
            pltpu.VMEM((_M,), jnp.int32),
            pltpu.VMEM((N,), jnp.float32),
            pltpu.VMEM((N,), jnp.float32),
            pltpu.VMEM((N,), jnp.float32),
            pltpu.VMEM((N,), jnp.float32),
            pltpu.VMEM((N,), jnp.int32),
            pltpu.VMEM((N,), jnp.float32),
            pltpu.VMEM((N,), jnp.float32),
            pltpu.VMEM((N,), jnp.float32),
            pltpu.VMEM((64,), jnp.float32),
            pltpu.SemaphoreType.DMA,
        ],
    )
    return f(s1p, s2p, nb_arr, tables, embT, bp)


def kernel(predicted_iv, node_embeddings, geometry, liquidity, edge_index_strike, edge_index_maturity):
    geomT = geometry.T
    ivT = predicted_iv.T
    liqT = liquidity.reshape(1, N)
    embT = node_embeddings.T

    tables = pl.pallas_call(
        _tables_body,
        out_shape=jax.ShapeDtypeStruct((8, N), jnp.float32),
    )(ivT, geomT, liqT, embT)

    eis = edge_index_strike.astype(jnp.int32)
    eim = edge_index_maturity.astype(jnp.int32)
    key1, key2, nb_arr, cal_part = _run_b1(eis, tables, eim)

    nblk = N // _RBLK
    best_put = pl.pallas_call(
        _argmin_body,
        grid=(nblk,),
        in_specs=[
            pl.BlockSpec((_RBLK, 3), lambda i: (i, 0)),
            pl.BlockSpec((3, N), lambda i: (0, 0)),
        ],
        out_specs=pl.BlockSpec((1, 1, _RBLK), lambda i: (i, 0, 0)),
        out_shape=jax.ShapeDtypeStruct((nblk, 1, _RBLK), jnp.int32),
    )(geometry, geomT).reshape(N)

    tv = tables[0]
    liq_w = tables[1]
    callf = tables[2]
    enorm = tables[3]
    nrm2 = tables[4]
    lm = tables[5]
    tau = tables[6]
    call_mask = callf > 0.5

    s1p, s2p = _sort32k(key1, key2)
    (b2_part,) = _run_b2(s1p, s2p, nb_arr, tables, embT, best_put)

    cal_num = jnp.sum(cal_part[:, 0:16])
    cal_cnt = jnp.sum(cal_part[:, 16:32])
    cal = cal_num / jnp.maximum(cal_cnt, 1.0)
    bf_num = jnp.sum(b2_part[:, 0:16])
    cvx_num = jnp.sum(b2_part[:, 16:32])
    cnt = jnp.sum(b2_part[:, 32:48])
    pcp_sum = jnp.sum(b2_part[:, 48:64])
    bf = bf_num / jnp.maximum(cnt, 1.0)
    cvx = cvx_num / jnp.maximum(cnt, 1.0)

    c_cnt = tables[7, 0]
    p_cnt = N - c_cnt
    pcp = jnp.where((c_cnt > 0) & (p_cnt > 0), pcp_sum / jnp.maximum(c_cnt, 1.0),
                    jnp.asarray(0.0, dtype=jnp.float32))
    return CAL_W * cal + BF_W * bf + PCP_W * pcp + CVX_W * cvx


# speedup vs baseline: 63.0706x; 1.0006x over previous
import jax
import jax.numpy as jnp
from jax.experimental import pallas as pl
from jax.experimental.pallas import tpu as pltpu

N = 4096
D = 64
E = 16384
EPS = 1e-06
CAL_W = 1.0
BF_W = 1.0
PCP_W = 0.5
CVX_W = 0.5
LIQ_SCALE = 1.0

_RBLK = 512
_M = 2 * E
_ROWS, _LANES = _M // 128, 128


def _partner_rows(x, jr):
    a = x.reshape(_ROWS // (2 * jr), 2, jr, _LANES)
    return jnp.concatenate([a[:, 1], a[:, 0]], axis=1).reshape(_ROWS, _LANES)


def _sort_body(ka_ref, kb_ref, oa_ref, ob_ref):
    ka = ka_ref[...]
    kb = kb_ref[...]
    row_i = jax.lax.broadcasted_iota(jnp.int32, (_ROWS, _LANES), 0)
    lane_i = jax.lax.broadcasted_iota(jnp.int32, (_ROWS, _LANES), 1)
    idx = row_i * _LANES + lane_i
    for m in range(1, 16):
        k = 1 << m
        asc = (idx & k) == 0
        for j in [1 << t for t in range(m - 1, -1, -1)]:
            if j >= _LANES:
                jr = j // _LANES
                pa = _partner_rows(ka, jr)
                pb = _partner_rows(kb, jr)
            else:
                lower = (lane_i & j) == 0
                jneg = _LANES - j
                pa = jnp.where(lower, pltpu.roll(ka, jneg, 1), pltpu.roll(ka, j, 1))
                pb = jnp.where(lower, pltpu.roll(kb, jneg, 1), pltpu.roll(kb, j, 1))
            ahi = ka >> 15
            phi = pa >> 15
            gt = (ahi > phi) | ((ahi == phi) & ((kb > pb) | ((kb == pb) & (ka > pa))))
            keep_min = ((idx & j) == 0) == asc
            take_p = gt ^ (~keep_min)
            ka = jnp.where(take_p, pa, ka)
            kb = jnp.where(take_p, pb, kb)
    oa_ref[0:_ROWS, :] = ka
    ob_ref[0:_ROWS, :] = kb
    oa_ref[_ROWS:_ROWS + 1, :] = jnp.full((1, _LANES), _SENT, jnp.int32)
    ob_ref[_ROWS:_ROWS + 1, :] = jnp.zeros((1, _LANES), jnp.int32)


def _sort32k(ka, kb):
    outs = pl.pallas_call(
        _sort_body,
        out_shape=[jax.ShapeDtypeStruct((_ROWS + 1, _LANES), jnp.int32)] * 2,
    )(ka.reshape(_ROWS, _LANES), kb.reshape(_ROWS, _LANES))
    return [o.reshape(_M + _LANES) for o in outs]


def _tables_body(ivT, geomT, liqT, embT, out_ref):
    lm = geomT[0:1, :]
    tau = geomT[1:2, :]
    cp = geomT[2:3, :]
    iv = ivT[0:1, :]
    tv = iv * iv * jnp.maximum(tau, EPS)
    liq_w = jax.nn.sigmoid(LIQ_SCALE * liqT[0:1, :])
    callf = (cp > 0.5).astype(jnp.float32)
    nrm2 = jnp.sum(embT[...] * embT[...], axis=0, keepdims=True)
    enorm = jnp.sqrt(nrm2)
    out_ref[0:1, :] = tv
    out_ref[1:2, :] = liq_w
    out_ref[2:3, :] = callf
    out_ref[3:4, :] = enorm
    out_ref[4:5, :] = nrm2
    out_ref[5:6, :] = lm
    out_ref[6:7, :] = tau
    out_ref[7:8, :] = jnp.full((1, N), jnp.sum(callf), jnp.float32)


def _argmin_body(rows_ref, colsT_ref, bp_ref):
    lm_r = rows_ref[:, 0:1]
    tau_r = rows_ref[:, 1:2]
    lm_c = colsT_ref[0:1, :]
    tau_c = colsT_ref[1:2, :]
    put_c = colsT_ref[2:3, :] <= 0.5
    d2 = (lm_r - lm_c) ** 2 + (tau_r - tau_c) ** 2
    d2m = jnp.where(put_c, d2, jnp.inf)
    bp = jnp.argmin(d2m, axis=1).astype(jnp.int32)
    bp_ref[0, 0, :] = bp


from jax.experimental.pallas import tpu_sc as plsc

_NW = 32
_CHUNK = _M // _NW
_CAL_CHUNK = E // _NW
_SENT = 0x7FFFFFFF


def _flat_tile_id():
    import jax.lax as lax
    return lax.axis_index("s") * 2 + lax.axis_index("c")


def _b1_body(eis_hbm, tab_hbm, eim_hbm, k1_hbm, k2_hbm, nbo_hbm, part_hbm,
             i_buf, nb_buf, cf_buf, lm_buf, tau_buf, tv_buf, lw_buf,
             k1_buf, k2_buf, sm_buf, dm_buf, out_buf, sem):
    w = _flat_tile_id()
    base = w * _CHUNK
    half = w // 16
    off = (w % 16) * _CHUNK
    cbase = w * _CAL_CHUNK
    copies = [
        pltpu.async_copy(eis_hbm.at[half, pl.ds(off, _CHUNK)], i_buf, sem),
        pltpu.async_copy(eis_hbm.at[1 - half, pl.ds(off, _CHUNK)], nb_buf, sem),
        pltpu.async_copy(tab_hbm.at[2], cf_buf, sem),
        pltpu.async_copy(tab_hbm.at[5], lm_buf, sem),
        pltpu.async_copy(tab_hbm.at[6], tau_buf, sem),
        pltpu.async_copy(tab_hbm.at[0], tv_buf, sem),
        pltpu.async_copy(tab_hbm.at[1], lw_buf, sem),
        pltpu.async_copy(eim_hbm.at[0, pl.ds(cbase, _CAL_CHUNK)], sm_buf, sem),
        pltpu.async_copy(eim_hbm.at[1, pl.ds(cbase, _CAL_CHUNK)], dm_buf, sem),
    ]
    for cp in copies:
        cp.wait()
    pltpu.sync_copy(nb_buf, nbo_hbm.at[pl.ds(base, _CHUNK)])

    iota16 = jax.lax.iota(jnp.int32, 16)

    def key_step(t, _):
        s = t * 16
        iv = i_buf[pl.ds(s, 16)]
        nv = nb_buf[pl.ds(s, 16)]
        ci = plsc.load_gather(cf_buf, [iv]) > 0.5
        cn = plsc.load_gather(cf_buf, [nv]) > 0.5
        inv = jnp.where(ci & cn, 0, 1)
        rank = base + s + iota16
        k1_buf[pl.ds(s, 16)] = ((iv * 2 + inv) << 15) | rank
        k2_buf[pl.ds(s, 16)] = plsc.bitcast(plsc.load_gather(lm_buf, [nv]),
                                            jnp.int32)
        return _

    jax.lax.fori_loop(0, _CHUNK // 16, key_step, 0)
    pltpu.sync_copy(k1_buf, k1_hbm.at[pl.ds(base, _CHUNK)])
    pltpu.sync_copy(k2_buf, k2_hbm.at[pl.ds(base, _CHUNK)])

    def cal_step(t, acc):
        acc_n, acc_c = acc
        s = t * 16
        sv = sm_buf[pl.ds(s, 16)]
        dv = dm_buf[pl.ds(s, 16)]
        ts = plsc.load_gather(tau_buf, [sv])
        td = plsc.load_gather(tau_buf, [dv])
        vcal = ts < td - EPS
        viol = jnp.maximum(plsc.load_gather(tv_buf, [sv])
                           - plsc.load_gather(tv_buf, [dv]), 0.0)
        wgt = jnp.minimum(plsc.load_gather(lw_buf, [sv]),
                          plsc.load_gather(lw_buf, [dv]))
        acc_n = acc_n + jnp.where(vcal, wgt * viol, 0.0)
        acc_c = acc_c + jnp.where(vcal, 1.0, 0.0)
        return acc_n, acc_c

    zero = jnp.zeros((16,), jnp.float32)
    cal_n, cal_c = jax.lax.fori_loop(0, _CAL_CHUNK // 16, cal_step, (zero, zero))
    out_buf[pl.ds(0, 16)] = cal_n
    out_buf[pl.ds(16, 16)] = cal_c
    pltpu.sync_copy(out_buf, part_hbm.at[w])


def _b2_body(s1_hbm, s2_hbm, nb_hbm, tab_hbm, embT_hbm, bp_hbm,
             part_hbm,
             s1_buf, s2_buf, nb_buf, tv_buf, en_buf, lw_buf, cf_buf,
             bp_buf, ed0_buf, ed1_buf, w_buf, out_buf, sem):
    w = _flat_tile_id()
    base = w * _CHUNK
    hl = _CHUNK + 16
    copies = [
        pltpu.async_copy(s1_hbm.at[pl.ds(base, hl)], s1_buf, sem),
        pltpu.async_copy(s2_hbm.at[pl.ds(base, hl)], s2_buf, sem),
        pltpu.async_copy(nb_hbm, nb_buf, sem),
        pltpu.async_copy(tab_hbm.at[0], tv_buf, sem),
        pltpu.async_copy(tab_hbm.at[3], en_buf, sem),
        pltpu.async_copy(tab_hbm.at[1], lw_buf, sem),
        pltpu.async_copy(tab_hbm.at[2], cf_buf, sem),
        pltpu.async_copy(bp_hbm, bp_buf, sem),
        pltpu.async_copy(embT_hbm.at[2 * w], ed0_buf, sem),
        pltpu.async_copy(embT_hbm.at[2 * w + 1], ed1_buf, sem),
    ]
    for cp in copies:
        cp.wait()

    iota16 = jax.lax.iota(jnp.int32, 16)

    def tri_step(t, acc):
        acc_bf, acc_cv, acc_ct = acc
        s = t * 16
        a = s1_buf[pl.ds(s, 16)]
        b = s1_buf[pl.ds(s + 1, 16)]
        c = s1_buf[pl.ds(s + 2, 16)]
        lm1 = plsc.bitcast(s2_buf[pl.ds(s, 16)], jnp.float32)
        lm2 = plsc.bitcast(s2_buf[pl.ds(s + 1, 16)], jnp.float32)
        lm3 = plsc.bitcast(s2_buf[pl.ds(s + 2, 16)], jnp.float32)
        sn1 = plsc.load_gather(nb_buf, [a & 32767])
        sn2 = plsc.load_gather(nb_buf, [b & 32767])
        sn3 = plsc.load_gather(nb_buf, [c & 32767])
        w1 = plsc.load_gather(tv_buf, [sn1])
        w2 = plsc.load_gather(tv_buf, [sn2])
        w3 = plsc.load_gather(tv_buf, [sn3])
        e1 = plsc.load_gather(en_buf, [sn1])
        e2 = plsc.load_gather(en_buf, [sn2])
        e3 = plsc.load_gather(en_buf, [sn3])
        lwn2 = plsc.load_gather(lw_buf, [sn2])
        lwi = plsc.load_gather(
            lw_buf, [jnp.minimum(jax.lax.shift_right_logical(a, 16), N - 1)])
        ahi = a >> 15
        bhi = b >> 15
        chi = c >> 15
        gj = base + s + iota16
        tmask = ((ahi == bhi) & (bhi == chi) & ((ahi & 1) == 0)
                 & (gj < (_M - 2)))
        denom = lm3 - lm1 + EPS
        alpha = (lm3 - lm2) / denom
        beta = (lm2 - lm1) / denom
        wgt = lwi * lwn2
        bft = wgt * jnp.maximum(alpha * w1 + beta * w3 - w2, 0.0)
        cvt = wgt * jnp.maximum(e2 - (alpha * e1 + (1.0 - alpha) * e3), 0.0)
        acc_bf = acc_bf + jnp.where(tmask, bft, 0.0)
        acc_cv = acc_cv + jnp.where(tmask, cvt, 0.0)
        acc_ct = acc_ct + jnp.where(tmask, 1.0, 0.0)
        return acc_bf, acc_cv, acc_ct

    zero = jnp.zeros((16,), jnp.float32)
    acc_bf, acc_cv, acc_ct = jax.lax.fori_loop(
        0, _CHUNK // 16, tri_step, (zero, zero, zero))

    def w_step(t, _):
        s = t * 16
        bpv = bp_buf[pl.ds(s, 16)]
        w_buf[pl.ds(s, 16)] = (cf_buf[pl.ds(s, 16)] * lw_buf[pl.ds(s, 16)]
                               * plsc.load_gather(lw_buf, [bpv]))
        return _

    jax.lax.fori_loop(0, N // 16, w_step, 0)

    def pcp_step(t, acc):
        s = t * 16
        bpv = bp_buf[pl.ds(s, 16)]
        wv = w_buf[pl.ds(s, 16)]
        d0 = ed0_buf[pl.ds(s, 16)] - plsc.load_gather(ed0_buf, [bpv])
        d1 = ed1_buf[pl.ds(s, 16)] - plsc.load_gather(ed1_buf, [bpv])
        return acc + wv * (d0 * d0 + d1 * d1)

    acc_pcp = jax.lax.fori_loop(0, N // 16, pcp_step, zero)

    out_buf[pl.ds(0, 16)] = acc_bf
    out_buf[pl.ds(16, 16)] = acc_cv
    out_buf[pl.ds(32, 16)] = acc_ct
    out_buf[pl.ds(48, 16)] = acc_pcp
    pltpu.sync_copy(out_buf, part_hbm.at[w])


def _run_b1(eis, tables, eim):
    mesh = plsc.VectorSubcoreMesh(core_axis_name="c", subcore_axis_name="s",
                                  num_cores=2, num_subcores=16)
    f = pl.kernel(
        _b1_body,
        out_type=[
            jax.ShapeDtypeStruct((_M,), jnp.int32),
            jax.ShapeDtypeStruct((_M,), jnp.int32),
            jax.ShapeDtypeStruct((_M,), jnp.int32),
            jax.ShapeDtypeStruct((_NW, 32), jnp.float32),
        ],
        mesh=mesh,
        compiler_params=pltpu.CompilerParams(needs_layout_passes=False),
        scratch_types=[
            pltpu.VMEM((_CHUNK,), jnp.int32),
            pltpu.VMEM((_CHUNK,), jnp.int32),
            pltpu.VMEM((N,), jnp.float32),
            pltpu.VMEM((N,), jnp.float32),
            pltpu.VMEM((N,), jnp.float32),
            pltpu.VMEM((N,), jnp.float32),
            pltpu.VMEM((N,), jnp.float32),
            pltpu.VMEM((_CHUNK,), jnp.int32),
            pltpu.VMEM((_CHUNK,), jnp.int32),
            pltpu.VMEM((_CAL_CHUNK,), jnp.int32),
            pltpu.VMEM((_CAL_CHUNK,), jnp.int32),
            pltpu.VMEM((32,), jnp.float32),
            pltpu.SemaphoreType.DMA,
        ],
    )
    return f(eis, tables, eim)


def _run_b2(s1p, s2p, nb_arr, tables, embT, bp):
    mesh = plsc.VectorSubcoreMesh(core_axis_name="c", subcore_axis_name="s",
                                  num_cores=2, num_subcores=16)
    f = pl.kernel(
        _b2_body,
        out_type=[jax.ShapeDtypeStruct((_NW, 64), jnp.float32)],
        mesh=mesh,
        compiler_params=pltpu.CompilerParams(needs_layout_passes=False),
        scratch_types=[
            pltpu.VMEM((_CHUNK + 16,), jnp.int32),
            pltpu.VMEM((_CHUNK + 16,), jnp.int32),
            pltpu.VMEM((_M,), jnp.int32),
            pltpu.VMEM((N,), jnp.float32),
            pltpu.VMEM((N,), jnp.float32),
            pltpu.VMEM((N,), jnp.float32),
            pltpu.VMEM((N,), jnp.float32),
            pltpu.VMEM((N,), jnp.int32),
            pltpu.VMEM((N,), jnp.float32),
            pltpu.VMEM((N,), jnp.float32),
            pltpu.VMEM((N,), jnp.float32),
            pltpu.VMEM((64,), jnp.float32),
            pltpu.SemaphoreType.DMA,
        ],
    )
    return f(s1p, s2p, nb_arr, tables, embT, bp)


def kernel(predicted_iv, node_embeddings, geometry, liquidity, edge_index_strike, edge_index_maturity):
    geomT = geometry.T
    ivT = predicted_iv.T
    liqT = liquidity.reshape(1, N)
    embT = node_embeddings.T

    tables = pl.pallas_call(
        _tables_body,
        out_shape=jax.ShapeDtypeStruct((8, N), jnp.float32),
    )(ivT, geomT, liqT, embT)

    eis = edge_index_strike.astype(jnp.int32)
    eim = edge_index_maturity.astype(jnp.int32)
    key1, key2, nb_arr, cal_part = _run_b1(eis, tables, eim)

    nblk = N // _RBLK
    best_put = pl.pallas_call(
        _argmin_body,
        grid=(nblk,),
        in_specs=[
            pl.BlockSpec((_RBLK, 3), lambda i: (i, 0)),
            pl.BlockSpec((3, N), lambda i: (0, 0)),
        ],
        out_specs=pl.BlockSpec((1, 1, _RBLK), lambda i: (i, 0, 0)),
        out_shape=jax.ShapeDtypeStruct((nblk, 1, _RBLK), jnp.int32),
    )(geometry, geomT).reshape(N)

    tv = tables[0]
    liq_w = tables[1]
    callf = tables[2]
    enorm = tables[3]
    nrm2 = tables[4]
    lm = tables[5]
    tau = tables[6]
    call_mask = callf > 0.5

    s1p, s2p = _sort32k(key1, key2)
    (b2_part,) = _run_b2(s1p, s2p, nb_arr, tables, embT, best_put)

    cal_num = jnp.sum(cal_part[:, 0:16])
    cal_cnt = jnp.sum(cal_part[:, 16:32])
    cal = cal_num / jnp.maximum(cal_cnt, 1.0)
    bf_num = jnp.sum(b2_part[:, 0:16])
    cvx_num = jnp.sum(b2_part[:, 16:32])
    cnt = jnp.sum(b2_part[:, 32:48])
    pcp_sum = jnp.sum(b2_part[:, 48:64])
    bf = bf_num / jnp.maximum(cnt, 1.0)
    cvx = cvx_num / jnp.maximum(cnt, 1.0)

    c_cnt = tables[7, 0]
    p_cnt = N - c_cnt
    pcp = jnp.where((c_cnt > 0) & (p_cnt > 0), pcp_sum / jnp.maximum(c_cnt, 1.0),
                    jnp.asarray(0.0, dtype=jnp.float32))
    return CAL_W * cal + BF_W * bf + PCP_W * pcp + CVX_W * cvx
